# baseline (speedup 1.0000x reference)
import functools
import math

import jax
import jax.numpy as jnp
from jax import lax
from jax.experimental import pallas as pl
from jax.experimental.pallas import tpu as pltpu
from jax.experimental.pallas import tpu_sc as plsc

_NC = 2
_NS = 16
_LANES = 16
_NW = _NC * _NS

_EPAD = 128
_NEG = -1e30


def _route_body(x0_ref, s_ref, b_ref, wr_ref, br_ref,
                x_ref, sd0_ref, sd1_ref, cd0_ref, cd1_ref, cg0_ref, cg1_ref,
                carry_ref, *, blk, n_exp, cap, ec):
    b = pl.program_id(0)

    @pl.when(b == 0)
    def _():
        carry_ref[...] = jnp.zeros_like(carry_ref)

    x0 = x0_ref[...]
    mean = jnp.mean(x0, axis=1, keepdims=True)
    var = jnp.mean((x0 - mean) ** 2, axis=1, keepdims=True)
    x = (x0 - mean) * lax.rsqrt(var + 1e-6) * s_ref[...] + b_ref[...]
    x = jnp.maximum(x, 0.0)
    x_ref[...] = x

    logits = jnp.dot(x, wr_ref[...], preferred_element_type=jnp.float32)
    logits = logits + br_ref[...]
    lane = lax.broadcasted_iota(jnp.int32, logits.shape, 1)
    logits = jnp.where(lane < n_exp, logits, _NEG)

    m1 = jnp.max(logits, axis=1, keepdims=True)
    e0 = jnp.min(jnp.where(logits == m1, lane, _EPAD), axis=1, keepdims=True)
    l2 = jnp.where(lane == e0, _NEG, logits)
    m2 = jnp.max(l2, axis=1, keepdims=True)
    e1 = jnp.min(jnp.where(l2 == m2, lane, _EPAD), axis=1, keepdims=True)
    g0 = 1.0 / (1.0 + jnp.exp(m2 - m1))
    g1 = 1.0 - g0

    oh0 = (lane == e0).astype(jnp.float32)
    oh1 = (lane == e1).astype(jnp.float32)
    oh = oh0 + oh1
    ri = lax.broadcasted_iota(jnp.int32, (blk, blk), 0)
    ci = lax.broadcasted_iota(jnp.int32, (blk, blk), 1)
    ltri = (ci < ri).astype(jnp.float32)
    before = jnp.dot(ltri, oh, preferred_element_type=jnp.float32)
    before = before + carry_ref[...]
    carry_ref[...] = carry_ref[...] + jnp.sum(oh, axis=0, keepdims=True)

    pos0 = jnp.sum(before * oh0, axis=1, keepdims=True).astype(jnp.int32)
    pos1 = jnp.sum((before + oh0) * oh1, axis=1, keepdims=True).astype(jnp.int32)
    tok = b * blk + lax.broadcasted_iota(jnp.int32, (blk, 1), 0)
    v0 = pos0 < cap
    v1 = pos1 < cap
    trash = ec + jnp.bitwise_and(tok, _TRASH - 1)
    sd0_ref[...] = jnp.where(v0, e0 * cap + pos0, trash)
    sd1_ref[...] = jnp.where(v1, e1 * cap + pos1, trash)
    cd0_ref[...] = e0 * cap + jnp.minimum(pos0, cap - 1)
    cd1_ref[...] = e1 * cap + jnp.minimum(pos1, cap - 1)
    cg0_ref[...] = jnp.broadcast_to(jnp.where(v0, g0, 0.0), (blk, _LANES))
    cg1_ref[...] = jnp.broadcast_to(jnp.where(v1, g1, 0.0), (blk, _LANES))


def _route(xf, ln0_scale, ln0_bias, wr_p, br_p, *, blk, n_exp, cap, ec):
    n, d = xf.shape
    grid = n // blk
    body = functools.partial(_route_body, blk=blk, n_exp=n_exp, cap=cap, ec=ec)
    col_i = jax.ShapeDtypeStruct((n, 1), jnp.int32)
    gate_f = jax.ShapeDtypeStruct((n, _LANES), jnp.float32)
    return pl.pallas_call(
        body,
        grid=(grid,),
        in_specs=[
            pl.BlockSpec((blk, d), lambda b: (b, 0)),
            pl.BlockSpec((1, d), lambda b: (0, 0)),
            pl.BlockSpec((1, d), lambda b: (0, 0)),
            pl.BlockSpec((d, _EPAD), lambda b: (0, 0)),
            pl.BlockSpec((1, _EPAD), lambda b: (0, 0)),
        ],
        out_specs=[
            pl.BlockSpec((blk, d), lambda b: (b, 0)),
        ] + [pl.BlockSpec((blk, 1), lambda b: (b, 0))] * 4
          + [pl.BlockSpec((blk, _LANES), lambda b: (b, 0))] * 2,
        out_shape=[jax.ShapeDtypeStruct((n, d), jnp.float32),
                   col_i, col_i, col_i, col_i, gate_f, gate_f],
        scratch_shapes=[pltpu.VMEM((1, _EPAD), jnp.float32)],
        compiler_params=pltpu.CompilerParams(
            dimension_semantics=("arbitrary",)),
    )(xf, ln0_scale, ln0_bias, wr_p, br_p)


_TRASH = 512


def _dispatch(xf, sd0, sd1, *, ec, n, d):
    tpw = n // _NW
    gch = 32
    nck = tpw // gch
    mesh = plsc.VectorSubcoreMesh(core_axis_name="c", subcore_axis_name="s")

    @functools.partial(
        pl.kernel,
        out_type=jax.ShapeDtypeStruct((ec + _TRASH, d), jnp.float32),
        mesh=mesh,
        scratch_types=[
            [pltpu.VMEM((gch,), jnp.int32)] * 2,
            [pltpu.VMEM((gch,), jnp.int32)] * 2,
            [pltpu.VMEM((gch, d), jnp.float32)] * 2,
            [pltpu.SemaphoreType.DMA] * 2,
            [pltpu.SemaphoreType.DMA] * 2,
            [pltpu.SemaphoreType.DMA] * 2,
        ],
        compiler_params=pltpu.CompilerParams(use_tc_tiling_on_sc=False),
    )
    def k(x_hbm, sd0_hbm, sd1_hbm, out_hbm, i0, i1, xbuf, xg, s0, s1):
        cid = lax.axis_index("c")
        sid = lax.axis_index("s")
        wid = sid * _NC + cid
        cps = [[None, None], [None, None], [None, None]]
        for c in range(nck):
            bb = c & 1
            if c >= 2:
                cps[1][bb].wait()
                cps[2][bb].wait()
            tokbase = wid * tpw + c * gch
            cps[0][bb] = pltpu.async_copy(
                x_hbm.at[pl.ds(tokbase, gch)], xbuf[bb], xg[bb])
            pltpu.sync_copy(sd0_hbm.at[pl.ds(tokbase, gch)], i0[bb])
            pltpu.sync_copy(sd1_hbm.at[pl.ds(tokbase, gch)], i1[bb])
            cps[0][bb].wait()
            cps[1][bb] = pltpu.async_copy(
                xbuf[bb], out_hbm.at[i0[bb]], s0[bb])
            cps[2][bb] = pltpu.async_copy(
                xbuf[bb], out_hbm.at[i1[bb]], s1[bb])
        for c in range(min(nck, 2)):
            cps[1][c].wait()
            cps[2][c].wait()

    return k(xf, sd0, sd1)


def _mlp_body(xe_ref, w1_ref, b1_ref, s1_ref, t1_ref, w2_ref, b2_ref, y_ref):
    xe = xe_ref[...]
    h = jnp.dot(xe, w1_ref[0], preferred_element_type=jnp.float32) + b1_ref[0]
    mean = jnp.mean(h, axis=1, keepdims=True)
    var = jnp.mean((h - mean) ** 2, axis=1, keepdims=True)
    h = (h - mean) * lax.rsqrt(var + 1e-6) * s1_ref[0] + t1_ref[0]
    h = jnp.maximum(h, 0.0)
    y_ref[...] = jnp.dot(h, w2_ref[0],
                         preferred_element_type=jnp.float32) + b2_ref[0]


def _mlp(ei, w1, b1, s1, t1, w2, b2, *, n_exp, cap, d, hdim):
    return pl.pallas_call(
        _mlp_body,
        grid=(n_exp,),
        in_specs=[
            pl.BlockSpec((cap, d), lambda e: (e, 0)),
            pl.BlockSpec((1, d, hdim), lambda e: (e, 0, 0)),
            pl.BlockSpec((1, 1, hdim), lambda e: (e, 0, 0)),
            pl.BlockSpec((1, 1, hdim), lambda e: (e, 0, 0)),
            pl.BlockSpec((1, 1, hdim), lambda e: (e, 0, 0)),
            pl.BlockSpec((1, hdim, d), lambda e: (e, 0, 0)),
            pl.BlockSpec((1, 1, d), lambda e: (e, 0, 0)),
        ],
        out_specs=pl.BlockSpec((cap, d), lambda e: (e, 0)),
        out_shape=jax.ShapeDtypeStruct((n_exp * cap, d), jnp.float32),
        compiler_params=pltpu.CompilerParams(
            dimension_semantics=("arbitrary",)),
    )(ei, w1, b1, s1, t1, w2, b2)


def _combine(y, xf, cd0, cd1, cg0, cg1, *, n, d):
    tpw = n // _NW
    ch = 16
    nch = tpw // ch
    nseg = d // _LANES
    mesh = plsc.VectorSubcoreMesh(core_axis_name="c", subcore_axis_name="s")

    @functools.partial(
        pl.kernel,
        out_type=jax.ShapeDtypeStruct((n, d), jnp.float32),
        mesh=mesh,
        scratch_types=[
            pltpu.VMEM((tpw,), jnp.int32),
            pltpu.VMEM((tpw,), jnp.int32),
            pltpu.VMEM((tpw, _LANES), jnp.float32),
            pltpu.VMEM((tpw, _LANES), jnp.float32),
            [pltpu.VMEM((ch, d), jnp.float32)] * 3,
            [pltpu.VMEM((ch, d), jnp.float32)] * 3,
            [pltpu.VMEM((ch, d), jnp.float32)] * 3,
            [pltpu.SemaphoreType.DMA] * 3,
            [pltpu.SemaphoreType.DMA] * 3,
            [pltpu.SemaphoreType.DMA] * 3,
            [pltpu.SemaphoreType.DMA] * 3,
        ],
        compiler_params=pltpu.CompilerParams(use_tc_tiling_on_sc=False),
    )
    def k(y_hbm, x0_hbm, cd0_hbm, cd1_hbm, cg0_hbm, cg1_hbm, out_hbm,
          idx0, idx1, g0v, g1v, r0, r1, xb, s0, s1, sx, sw):
        cid = lax.axis_index("c")
        sid = lax.axis_index("s")
        wid = sid * _NC + cid
        base = wid * tpw
        pltpu.sync_copy(cd0_hbm.at[pl.ds(base, tpw)], idx0)
        pltpu.sync_copy(cd1_hbm.at[pl.ds(base, tpw)], idx1)
        pltpu.sync_copy(cg0_hbm.at[pl.ds(base, tpw)], g0v)
        pltpu.sync_copy(cg1_hbm.at[pl.ds(base, tpw)], g1v)

        nbuf = 3
        cp0 = [None] * nbuf
        cp1 = [None] * nbuf
        cpx = [None] * nbuf
        cpw = [None] * nbuf

        def stage(c):
            bb = c % nbuf
            cp0[bb] = pltpu.async_copy(
                y_hbm.at[idx0.at[pl.ds(c * ch, ch)]], r0[bb], s0[bb])
            cp1[bb] = pltpu.async_copy(
                y_hbm.at[idx1.at[pl.ds(c * ch, ch)]], r1[bb], s1[bb])
            cpx[bb] = pltpu.async_copy(
                x0_hbm.at[pl.ds(base + c * ch, ch)], xb[bb], sx[bb])

        def compute(c):
            bb = c % nbuf
            off = c * ch

            def row(j, _):
                gb0 = g0v[off + j, :]
                gb1 = g1v[off + j, :]
                for q in range(nseg):
                    seg = pl.ds(q * _LANES, _LANES)
                    r0[bb][j, seg] = (xb[bb][j, seg] + gb0 * r0[bb][j, seg]
                                      + gb1 * r1[bb][j, seg])
                return 0

            lax.fori_loop(0, ch, row, 0)

        stage(0)
        stage(1)
        for c in range(nch):
            bb = c % nbuf
            if c + 2 < nch:
                nb = (c + 2) % nbuf
                if c >= 1:
                    cpw[nb].wait()
                stage(c + 2)
            cp0[bb].wait()
            cp1[bb].wait()
            cpx[bb].wait()
            compute(c)
            cpw[bb] = pltpu.async_copy(
                r0[bb], out_hbm.at[pl.ds(base + c * ch, ch)], sw[bb])
        for c in range(max(0, nch - nbuf), nch):
            cpw[c % nbuf].wait()

    return k(y, xf, cd0, cd1, cg0, cg1)


def kernel(x0, ln0_scale, ln0_bias, Wr, br, W1, b1, ln1_scale, ln1_bias, W2, b2):
    B, S, D = x0.shape
    E = Wr.shape[-1]
    H = W1.shape[-1]
    N = B * S
    top_k = 2
    cap = max(1, int(math.ceil(1.0 * N * top_k / E)))
    ec = E * cap

    xf = x0.reshape(N, D)
    wr_p = jnp.zeros((D, _EPAD), jnp.float32).at[:, :E].set(Wr)
    br_p = jnp.zeros((1, _EPAD), jnp.float32).at[0, :E].set(br)

    x, sd0, sd1, cd0, cd1, cg0, cg1 = _route(
        xf, ln0_scale.reshape(1, D), ln0_bias.reshape(1, D), wr_p, br_p,
        blk=256, n_exp=E, cap=cap, ec=ec)

    ei = _dispatch(x, sd0.reshape(N), sd1.reshape(N), ec=ec, n=N, d=D)

    y = _mlp(ei, W1, b1.reshape(E, 1, H),
             ln1_scale.reshape(E, 1, H), ln1_bias.reshape(E, 1, H),
             W2, b2.reshape(E, 1, D), n_exp=E, cap=cap, d=D, hdim=H)

    out = _combine(y, xf, cd0.reshape(N), cd1.reshape(N),
                   cg0, cg1, n=N, d=D)

    return out.reshape(B, S, D)


# speedup vs baseline: 1.1722x; 1.1722x over previous
import functools
import math

import jax
import jax.numpy as jnp
from jax import lax
from jax.experimental import pallas as pl
from jax.experimental.pallas import tpu as pltpu
from jax.experimental.pallas import tpu_sc as plsc

_NC = 2
_NS = 16
_LANES = 16
_NW = _NC * _NS

_EPAD = 128
_NEG = -1e30


def _route_body(x0_ref, s_ref, b_ref, wr_ref, br_ref,
                x_ref, sd0_ref, sd1_ref, cd0_ref, cd1_ref, cg0_ref, cg1_ref,
                carry_ref, *, blk, n_exp, cap, ec):
    b = pl.program_id(0)

    @pl.when(b == 0)
    def _():
        carry_ref[...] = jnp.zeros_like(carry_ref)

    x0 = x0_ref[...]
    mean = jnp.mean(x0, axis=1, keepdims=True)
    var = jnp.mean((x0 - mean) ** 2, axis=1, keepdims=True)
    x = (x0 - mean) * lax.rsqrt(var + 1e-6) * s_ref[...] + b_ref[...]
    x = jnp.maximum(x, 0.0)
    x_ref[...] = x

    logits = jnp.dot(x, wr_ref[...], preferred_element_type=jnp.float32)
    logits = logits + br_ref[...]
    lane = lax.broadcasted_iota(jnp.int32, logits.shape, 1)
    logits = jnp.where(lane < n_exp, logits, _NEG)

    m1 = jnp.max(logits, axis=1, keepdims=True)
    e0 = jnp.min(jnp.where(logits == m1, lane, _EPAD), axis=1, keepdims=True)
    l2 = jnp.where(lane == e0, _NEG, logits)
    m2 = jnp.max(l2, axis=1, keepdims=True)
    e1 = jnp.min(jnp.where(l2 == m2, lane, _EPAD), axis=1, keepdims=True)
    g0 = 1.0 / (1.0 + jnp.exp(m2 - m1))
    g1 = 1.0 - g0

    oh0 = (lane == e0).astype(jnp.float32)
    oh1 = (lane == e1).astype(jnp.float32)
    oh = oh0 + oh1
    ri = lax.broadcasted_iota(jnp.int32, (blk, blk), 0)
    ci = lax.broadcasted_iota(jnp.int32, (blk, blk), 1)
    ltri = (ci < ri).astype(jnp.float32)
    before = jnp.dot(ltri, oh, preferred_element_type=jnp.float32)
    before = before + carry_ref[...]
    carry_ref[...] = carry_ref[...] + jnp.sum(oh, axis=0, keepdims=True)

    pos0 = jnp.sum(before * oh0, axis=1, keepdims=True).astype(jnp.int32)
    pos1 = jnp.sum((before + oh0) * oh1, axis=1, keepdims=True).astype(jnp.int32)
    tok = b * blk + lax.broadcasted_iota(jnp.int32, (blk, 1), 0)
    v0 = pos0 < cap
    v1 = pos1 < cap
    trash = ec + jnp.bitwise_and(tok, _TRASH - 1)
    sd0_ref[...] = jnp.where(v0, e0 * cap + pos0, trash)
    sd1_ref[...] = jnp.where(v1, e1 * cap + pos1, trash)
    cd0_ref[...] = e0 * cap + jnp.minimum(pos0, cap - 1)
    cd1_ref[...] = e1 * cap + jnp.minimum(pos1, cap - 1)
    cg0_ref[...] = jnp.broadcast_to(jnp.where(v0, g0, 0.0), (blk, _LANES))
    cg1_ref[...] = jnp.broadcast_to(jnp.where(v1, g1, 0.0), (blk, _LANES))


def _route(xf, ln0_scale, ln0_bias, wr_p, br_p, *, blk, n_exp, cap, ec):
    n, d = xf.shape
    grid = n // blk
    body = functools.partial(_route_body, blk=blk, n_exp=n_exp, cap=cap, ec=ec)
    col_i = jax.ShapeDtypeStruct((n, 1), jnp.int32)
    gate_f = jax.ShapeDtypeStruct((n, _LANES), jnp.float32)
    return pl.pallas_call(
        body,
        grid=(grid,),
        in_specs=[
            pl.BlockSpec((blk, d), lambda b: (b, 0)),
            pl.BlockSpec((1, d), lambda b: (0, 0)),
            pl.BlockSpec((1, d), lambda b: (0, 0)),
            pl.BlockSpec((d, _EPAD), lambda b: (0, 0)),
            pl.BlockSpec((1, _EPAD), lambda b: (0, 0)),
        ],
        out_specs=[
            pl.BlockSpec((blk, d), lambda b: (b, 0)),
        ] + [pl.BlockSpec((blk, 1), lambda b: (b, 0))] * 4
          + [pl.BlockSpec((blk, _LANES), lambda b: (b, 0))] * 2,
        out_shape=[jax.ShapeDtypeStruct((n, d), jnp.float32),
                   col_i, col_i, col_i, col_i, gate_f, gate_f],
        scratch_shapes=[pltpu.VMEM((1, _EPAD), jnp.float32)],
        compiler_params=pltpu.CompilerParams(
            dimension_semantics=("arbitrary",)),
    )(xf, ln0_scale, ln0_bias, wr_p, br_p)


_TRASH = 512


def _dispatch(xf, sd0, sd1, *, ec, n, d):
    tpw = n // _NW
    gch = 32
    nck = tpw // gch
    mesh = plsc.VectorSubcoreMesh(core_axis_name="c", subcore_axis_name="s")

    @functools.partial(
        pl.kernel,
        out_type=jax.ShapeDtypeStruct((ec + _TRASH, d), jnp.float32),
        mesh=mesh,
        scratch_types=[
            [pltpu.VMEM((gch,), jnp.int32)] * 2,
            [pltpu.VMEM((gch,), jnp.int32)] * 2,
            [pltpu.VMEM((gch, d), jnp.float32)] * 2,
            [pltpu.SemaphoreType.DMA] * 2,
            [pltpu.SemaphoreType.DMA] * 2,
            [pltpu.SemaphoreType.DMA] * 2,
        ],
        compiler_params=pltpu.CompilerParams(use_tc_tiling_on_sc=False),
    )
    def k(x_hbm, sd0_hbm, sd1_hbm, out_hbm, i0, i1, xbuf, xg, s0, s1):
        cid = lax.axis_index("c")
        sid = lax.axis_index("s")
        wid = sid * _NC + cid
        cps = [[None, None], [None, None], [None, None]]
        for c in range(nck):
            bb = c & 1
            if c >= 2:
                cps[1][bb].wait()
                cps[2][bb].wait()
            tokbase = wid * tpw + c * gch
            cps[0][bb] = pltpu.async_copy(
                x_hbm.at[pl.ds(tokbase, gch)], xbuf[bb], xg[bb])
            pltpu.sync_copy(sd0_hbm.at[pl.ds(tokbase, gch)], i0[bb])
            pltpu.sync_copy(sd1_hbm.at[pl.ds(tokbase, gch)], i1[bb])
            cps[0][bb].wait()
            cps[1][bb] = pltpu.async_copy(
                xbuf[bb], out_hbm.at[i0[bb]], s0[bb])
            cps[2][bb] = pltpu.async_copy(
                xbuf[bb], out_hbm.at[i1[bb]], s1[bb])
        for c in range(min(nck, 2)):
            cps[1][c].wait()
            cps[2][c].wait()

    return k(xf, sd0, sd1)


def _mlp_body(xe_ref, w1_ref, b1_ref, s1_ref, t1_ref, w2_ref, b2_ref, y_ref,
              *, cap, d):
    xe = jnp.reshape(xe_ref[...], (cap, d))
    h = jnp.dot(xe, w1_ref[0], preferred_element_type=jnp.float32) + b1_ref[0]
    mean = jnp.mean(h, axis=1, keepdims=True)
    var = jnp.mean((h - mean) ** 2, axis=1, keepdims=True)
    h = (h - mean) * lax.rsqrt(var + 1e-6) * s1_ref[0] + t1_ref[0]
    h = jnp.maximum(h, 0.0)
    y = jnp.dot(h, w2_ref[0], preferred_element_type=jnp.float32) + b2_ref[0]
    y_ref[...] = jnp.reshape(y, y_ref.shape)


def _mlp(ei28, w1, b1, s1, t1, w2, b2, *, n_exp, cap, d, hdim):
    fold = d // 128
    body = functools.partial(_mlp_body, cap=cap, d=d)
    return pl.pallas_call(
        body,
        grid=(n_exp,),
        in_specs=[
            pl.BlockSpec((cap * fold, 128), lambda e: (e, 0)),
            pl.BlockSpec((1, d, hdim), lambda e: (e, 0, 0)),
            pl.BlockSpec((1, 1, hdim), lambda e: (e, 0, 0)),
            pl.BlockSpec((1, 1, hdim), lambda e: (e, 0, 0)),
            pl.BlockSpec((1, 1, hdim), lambda e: (e, 0, 0)),
            pl.BlockSpec((1, hdim, d), lambda e: (e, 0, 0)),
            pl.BlockSpec((1, 1, d), lambda e: (e, 0, 0)),
        ],
        out_specs=pl.BlockSpec((cap * fold, 128), lambda e: (e, 0)),
        out_shape=jax.ShapeDtypeStruct((n_exp * cap * fold, 128), jnp.float32),
        compiler_params=pltpu.CompilerParams(
            dimension_semantics=("arbitrary",)),
    )(ei28, w1, b1, s1, t1, w2, b2)


def _combine(y, xf, cd0, cd1, cg0, cg1, *, n, d):
    tpw = n // _NW
    ch = 16
    nch = tpw // ch
    nseg = d // _LANES
    mesh = plsc.VectorSubcoreMesh(core_axis_name="c", subcore_axis_name="s")

    @functools.partial(
        pl.kernel,
        out_type=jax.ShapeDtypeStruct((n, d), jnp.float32),
        mesh=mesh,
        scratch_types=[
            pltpu.VMEM((tpw,), jnp.int32),
            pltpu.VMEM((tpw,), jnp.int32),
            pltpu.VMEM((tpw, _LANES), jnp.float32),
            pltpu.VMEM((tpw, _LANES), jnp.float32),
            [pltpu.VMEM((ch, d), jnp.float32)] * 3,
            [pltpu.VMEM((ch, d), jnp.float32)] * 3,
            [pltpu.VMEM((ch, d), jnp.float32)] * 3,
            [pltpu.SemaphoreType.DMA] * 3,
            [pltpu.SemaphoreType.DMA] * 3,
            [pltpu.SemaphoreType.DMA] * 3,
            [pltpu.SemaphoreType.DMA] * 3,
        ],
        compiler_params=pltpu.CompilerParams(use_tc_tiling_on_sc=False),
    )
    def k(y_hbm, x0_hbm, cd0_hbm, cd1_hbm, cg0_hbm, cg1_hbm, out_hbm,
          idx0, idx1, g0v, g1v, r0, r1, xb, s0, s1, sx, sw):
        cid = lax.axis_index("c")
        sid = lax.axis_index("s")
        wid = sid * _NC + cid
        base = wid * tpw
        pltpu.sync_copy(cd0_hbm.at[pl.ds(base, tpw)], idx0)
        pltpu.sync_copy(cd1_hbm.at[pl.ds(base, tpw)], idx1)
        pltpu.sync_copy(cg0_hbm.at[pl.ds(base, tpw)], g0v)
        pltpu.sync_copy(cg1_hbm.at[pl.ds(base, tpw)], g1v)

        nbuf = 3
        cp0 = [None] * nbuf
        cp1 = [None] * nbuf
        cpx = [None] * nbuf
        cpw = [None] * nbuf

        def stage(c):
            bb = c % nbuf
            cp0[bb] = pltpu.async_copy(
                y_hbm.at[idx0.at[pl.ds(c * ch, ch)]], r0[bb], s0[bb])
            cp1[bb] = pltpu.async_copy(
                y_hbm.at[idx1.at[pl.ds(c * ch, ch)]], r1[bb], s1[bb])
            cpx[bb] = pltpu.async_copy(
                x0_hbm.at[pl.ds(base + c * ch, ch)], xb[bb], sx[bb])

        def compute(c):
            bb = c % nbuf
            off = c * ch

            def row(j, _):
                gb0 = g0v[off + j, :]
                gb1 = g1v[off + j, :]
                for q in range(nseg):
                    seg = pl.ds(q * _LANES, _LANES)
                    r0[bb][j, seg] = (xb[bb][j, seg] + gb0 * r0[bb][j, seg]
                                      + gb1 * r1[bb][j, seg])
                return 0

            lax.fori_loop(0, ch, row, 0)

        stage(0)
        stage(1)
        for c in range(nch):
            bb = c % nbuf
            if c + 2 < nch:
                nb = (c + 2) % nbuf
                if c >= 1:
                    cpw[nb].wait()
                stage(c + 2)
            cp0[bb].wait()
            cp1[bb].wait()
            cpx[bb].wait()
            compute(c)
            cpw[bb] = pltpu.async_copy(
                r0[bb], out_hbm.at[pl.ds(base + c * ch, ch)], sw[bb])
        for c in range(max(0, nch - nbuf), nch):
            cpw[c % nbuf].wait()

    return k(y, xf, cd0, cd1, cg0, cg1)


def kernel(x0, ln0_scale, ln0_bias, Wr, br, W1, b1, ln1_scale, ln1_bias, W2, b2):
    B, S, D = x0.shape
    E = Wr.shape[-1]
    H = W1.shape[-1]
    N = B * S
    top_k = 2
    cap = max(1, int(math.ceil(1.0 * N * top_k / E)))
    ec = E * cap

    xf = x0.reshape(N, D)
    wr_p = jnp.zeros((D, _EPAD), jnp.float32).at[:, :E].set(Wr)
    br_p = jnp.zeros((1, _EPAD), jnp.float32).at[0, :E].set(br)

    x, sd0, sd1, cd0, cd1, cg0, cg1 = _route(
        xf, ln0_scale.reshape(1, D), ln0_bias.reshape(1, D), wr_p, br_p,
        blk=256, n_exp=E, cap=cap, ec=ec)

    ei = _dispatch(x, sd0.reshape(N), sd1.reshape(N), ec=ec, n=N, d=D)

    fold = D // 128
    y24 = _mlp(ei.reshape((ec + _TRASH) * fold, 128), W1, b1.reshape(E, 1, H),
               ln1_scale.reshape(E, 1, H), ln1_bias.reshape(E, 1, H),
               W2, b2.reshape(E, 1, D), n_exp=E, cap=cap, d=D, hdim=H)

    out = _combine(y24.reshape(ec, D), xf, cd0.reshape(N), cd1.reshape(N),
                   cg0, cg1, n=N, d=D)

    return out.reshape(B, S, D)


# speedup vs baseline: 1.1957x; 1.0200x over previous
import functools
import math

import jax
import jax.numpy as jnp
from jax import lax
from jax.experimental import pallas as pl
from jax.experimental.pallas import tpu as pltpu
from jax.experimental.pallas import tpu_sc as plsc

_NC = 2
_NS = 16
_LANES = 16
_NW = _NC * _NS

_EPAD = 128
_NEG = -1e30


def _route_body(x0_ref, s_ref, b_ref, wr_ref, br_ref,
                x_ref, sd0_ref, sd1_ref, cd0_ref, cd1_ref, cg0_ref, cg1_ref,
                carry_ref, *, blk, n_exp, cap, ec):
    b = pl.program_id(0)

    @pl.when(b == 0)
    def _():
        carry_ref[...] = jnp.zeros_like(carry_ref)

    x0 = x0_ref[...]
    mean = jnp.mean(x0, axis=1, keepdims=True)
    var = jnp.mean((x0 - mean) ** 2, axis=1, keepdims=True)
    x = (x0 - mean) * lax.rsqrt(var + 1e-6) * s_ref[...] + b_ref[...]
    x = jnp.maximum(x, 0.0)
    x_ref[...] = jnp.reshape(x, x_ref.shape)

    logits = jnp.dot(x, wr_ref[...], preferred_element_type=jnp.float32)
    logits = logits + br_ref[...]
    lane = lax.broadcasted_iota(jnp.int32, logits.shape, 1)
    logits = jnp.where(lane < n_exp, logits, _NEG)

    m1 = jnp.max(logits, axis=1, keepdims=True)
    e0 = jnp.min(jnp.where(logits == m1, lane, _EPAD), axis=1, keepdims=True)
    l2 = jnp.where(lane == e0, _NEG, logits)
    m2 = jnp.max(l2, axis=1, keepdims=True)
    e1 = jnp.min(jnp.where(l2 == m2, lane, _EPAD), axis=1, keepdims=True)
    g0 = 1.0 / (1.0 + jnp.exp(m2 - m1))
    g1 = 1.0 - g0

    oh0 = (lane == e0).astype(jnp.float32)
    oh1 = (lane == e1).astype(jnp.float32)
    oh = oh0 + oh1
    ri = lax.broadcasted_iota(jnp.int32, (blk, blk), 0)
    ci = lax.broadcasted_iota(jnp.int32, (blk, blk), 1)
    ltri = (ci < ri).astype(jnp.float32)
    before = jnp.dot(ltri, oh, preferred_element_type=jnp.float32)
    before = before + carry_ref[...]
    carry_ref[...] = carry_ref[...] + jnp.sum(oh, axis=0, keepdims=True)

    pos0 = jnp.sum(before * oh0, axis=1, keepdims=True).astype(jnp.int32)
    pos1 = jnp.sum((before + oh0) * oh1, axis=1, keepdims=True).astype(jnp.int32)
    tok = b * blk + lax.broadcasted_iota(jnp.int32, (blk, 1), 0)
    v0 = pos0 < cap
    v1 = pos1 < cap
    trash = ec + jnp.bitwise_and(tok, _TRASH - 1)
    sd0_ref[...] = jnp.where(v0, e0 * cap + pos0, trash)
    sd1_ref[...] = jnp.where(v1, e1 * cap + pos1, trash)
    cd0_ref[...] = e0 * cap + jnp.minimum(pos0, cap - 1)
    cd1_ref[...] = e1 * cap + jnp.minimum(pos1, cap - 1)
    cg0_ref[...] = jnp.broadcast_to(jnp.where(v0, g0, 0.0), (blk, _LANES))
    cg1_ref[...] = jnp.broadcast_to(jnp.where(v1, g1, 0.0), (blk, _LANES))


def _route(xf, ln0_scale, ln0_bias, wr_p, br_p, *, blk, n_exp, cap, ec):
    n, d = xf.shape
    grid = n // blk
    body = functools.partial(_route_body, blk=blk, n_exp=n_exp, cap=cap, ec=ec)
    col_i = jax.ShapeDtypeStruct((n, 1), jnp.int32)
    gate_f = jax.ShapeDtypeStruct((n, _LANES), jnp.float32)
    return pl.pallas_call(
        body,
        grid=(grid,),
        in_specs=[
            pl.BlockSpec((blk, d), lambda b: (b, 0)),
            pl.BlockSpec((1, d), lambda b: (0, 0)),
            pl.BlockSpec((1, d), lambda b: (0, 0)),
            pl.BlockSpec((d, _EPAD), lambda b: (0, 0)),
            pl.BlockSpec((1, _EPAD), lambda b: (0, 0)),
        ],
        out_specs=[
            pl.BlockSpec((blk * (d // 128), 128), lambda b: (b, 0)),
        ] + [pl.BlockSpec((blk, 1), lambda b: (b, 0))] * 4
          + [pl.BlockSpec((blk, _LANES), lambda b: (b, 0))] * 2,
        out_shape=[jax.ShapeDtypeStruct((n * (d // 128), 128), jnp.float32),
                   col_i, col_i, col_i, col_i, gate_f, gate_f],
        scratch_shapes=[pltpu.VMEM((1, _EPAD), jnp.float32)],
        compiler_params=pltpu.CompilerParams(
            dimension_semantics=("arbitrary",)),
    )(xf, ln0_scale, ln0_bias, wr_p, br_p)


_TRASH = 512


def _dispatch(xf, sd0, sd1, *, ec, n, d):
    tpw = n // _NW
    gch = 32
    nck = tpw // gch
    mesh = plsc.VectorSubcoreMesh(core_axis_name="c", subcore_axis_name="s")

    @functools.partial(
        pl.kernel,
        out_type=jax.ShapeDtypeStruct((ec + _TRASH, d), jnp.float32),
        mesh=mesh,
        scratch_types=[
            [pltpu.VMEM((gch,), jnp.int32)] * 2,
            [pltpu.VMEM((gch,), jnp.int32)] * 2,
            [pltpu.VMEM((gch, d), jnp.float32)] * 2,
            [pltpu.SemaphoreType.DMA] * 2,
            [pltpu.SemaphoreType.DMA] * 2,
            [pltpu.SemaphoreType.DMA] * 2,
        ],
        compiler_params=pltpu.CompilerParams(use_tc_tiling_on_sc=False),
    )
    def k(x_hbm, sd0_hbm, sd1_hbm, out_hbm, i0, i1, xbuf, xg, s0, s1):
        cid = lax.axis_index("c")
        sid = lax.axis_index("s")
        wid = sid * _NC + cid
        cps = [[None, None], [None, None], [None, None]]
        for c in range(nck):
            bb = c & 1
            if c >= 2:
                cps[1][bb].wait()
                cps[2][bb].wait()
            tokbase = wid * tpw + c * gch
            cps[0][bb] = pltpu.async_copy(
                x_hbm.at[pl.ds(tokbase, gch)], xbuf[bb], xg[bb])
            pltpu.sync_copy(sd0_hbm.at[pl.ds(tokbase, gch)], i0[bb])
            pltpu.sync_copy(sd1_hbm.at[pl.ds(tokbase, gch)], i1[bb])
            cps[0][bb].wait()
            cps[1][bb] = pltpu.async_copy(
                xbuf[bb], out_hbm.at[i0[bb]], s0[bb])
            cps[2][bb] = pltpu.async_copy(
                xbuf[bb], out_hbm.at[i1[bb]], s1[bb])
        for c in range(min(nck, 2)):
            cps[1][c].wait()
            cps[2][c].wait()

    return k(xf, sd0, sd1)


def _mlp_body(xe_ref, w1_ref, b1_ref, s1_ref, t1_ref, w2_ref, b2_ref, y_ref,
              *, cap, d):
    xe = jnp.reshape(xe_ref[...], (cap, d))
    h = jnp.dot(xe, w1_ref[0], preferred_element_type=jnp.float32) + b1_ref[0]
    mean = jnp.mean(h, axis=1, keepdims=True)
    var = jnp.mean((h - mean) ** 2, axis=1, keepdims=True)
    h = (h - mean) * lax.rsqrt(var + 1e-6) * s1_ref[0] + t1_ref[0]
    h = jnp.maximum(h, 0.0)
    y = jnp.dot(h, w2_ref[0], preferred_element_type=jnp.float32) + b2_ref[0]
    y_ref[...] = jnp.reshape(y, y_ref.shape)


def _mlp(ei28, w1, b1, s1, t1, w2, b2, *, n_exp, cap, d, hdim):
    fold = d // 128
    body = functools.partial(_mlp_body, cap=cap, d=d)
    return pl.pallas_call(
        body,
        grid=(n_exp,),
        in_specs=[
            pl.BlockSpec((cap * fold, 128), lambda e: (e, 0)),
            pl.BlockSpec((1, d, hdim), lambda e: (e, 0, 0)),
            pl.BlockSpec((1, 1, hdim), lambda e: (e, 0, 0)),
            pl.BlockSpec((1, 1, hdim), lambda e: (e, 0, 0)),
            pl.BlockSpec((1, 1, hdim), lambda e: (e, 0, 0)),
            pl.BlockSpec((1, hdim, d), lambda e: (e, 0, 0)),
            pl.BlockSpec((1, 1, d), lambda e: (e, 0, 0)),
        ],
        out_specs=pl.BlockSpec((cap * fold, 128), lambda e: (e, 0)),
        out_shape=jax.ShapeDtypeStruct((n_exp * cap * fold, 128), jnp.float32),
        compiler_params=pltpu.CompilerParams(
            dimension_semantics=("arbitrary",)),
    )(ei28, w1, b1, s1, t1, w2, b2)


def _combine(y, xf, cd0, cd1, cg0, cg1, *, n, d):
    tpw = n // _NW
    ch = 16
    nch = tpw // ch
    nseg = d // _LANES
    mesh = plsc.VectorSubcoreMesh(core_axis_name="c", subcore_axis_name="s")

    @functools.partial(
        pl.kernel,
        out_type=jax.ShapeDtypeStruct((n, d), jnp.float32),
        mesh=mesh,
        scratch_types=[
            pltpu.VMEM((tpw,), jnp.int32),
            pltpu.VMEM((tpw,), jnp.int32),
            pltpu.VMEM((tpw, _LANES), jnp.float32),
            pltpu.VMEM((tpw, _LANES), jnp.float32),
            [pltpu.VMEM((ch, d), jnp.float32)] * 3,
            [pltpu.VMEM((ch, d), jnp.float32)] * 3,
            [pltpu.VMEM((ch, d), jnp.float32)] * 3,
            [pltpu.SemaphoreType.DMA] * 3,
            [pltpu.SemaphoreType.DMA] * 3,
            [pltpu.SemaphoreType.DMA] * 3,
            [pltpu.SemaphoreType.DMA] * 3,
        ],
        compiler_params=pltpu.CompilerParams(use_tc_tiling_on_sc=False),
    )
    def k(y_hbm, x0_hbm, cd0_hbm, cd1_hbm, cg0_hbm, cg1_hbm, out_hbm,
          idx0, idx1, g0v, g1v, r0, r1, xb, s0, s1, sx, sw):
        cid = lax.axis_index("c")
        sid = lax.axis_index("s")
        wid = sid * _NC + cid
        base = wid * tpw
        pltpu.sync_copy(cd0_hbm.at[pl.ds(base, tpw)], idx0)
        pltpu.sync_copy(cd1_hbm.at[pl.ds(base, tpw)], idx1)
        pltpu.sync_copy(cg0_hbm.at[pl.ds(base, tpw)], g0v)
        pltpu.sync_copy(cg1_hbm.at[pl.ds(base, tpw)], g1v)

        nbuf = 3
        cp0 = [None] * nbuf
        cp1 = [None] * nbuf
        cpx = [None] * nbuf
        cpw = [None] * nbuf

        def stage(c):
            bb = c % nbuf
            cp0[bb] = pltpu.async_copy(
                y_hbm.at[idx0.at[pl.ds(c * ch, ch)]], r0[bb], s0[bb])
            cp1[bb] = pltpu.async_copy(
                y_hbm.at[idx1.at[pl.ds(c * ch, ch)]], r1[bb], s1[bb])
            cpx[bb] = pltpu.async_copy(
                x0_hbm.at[pl.ds(base + c * ch, ch)], xb[bb], sx[bb])

        def compute(c):
            bb = c % nbuf
            off = c * ch

            def row(j, _):
                gb0 = g0v[off + j, :]
                gb1 = g1v[off + j, :]
                for q in range(nseg):
                    seg = pl.ds(q * _LANES, _LANES)
                    r0[bb][j, seg] = (xb[bb][j, seg] + gb0 * r0[bb][j, seg]
                                      + gb1 * r1[bb][j, seg])
                return 0

            lax.fori_loop(0, ch, row, 0)

        stage(0)
        stage(1)
        for c in range(nch):
            bb = c % nbuf
            if c + 2 < nch:
                nb = (c + 2) % nbuf
                if c >= 1:
                    cpw[nb].wait()
                stage(c + 2)
            cp0[bb].wait()
            cp1[bb].wait()
            cpx[bb].wait()
            compute(c)
            cpw[bb] = pltpu.async_copy(
                r0[bb], out_hbm.at[pl.ds(base + c * ch, ch)], sw[bb])
        for c in range(max(0, nch - nbuf), nch):
            cpw[c % nbuf].wait()

    return k(y, xf, cd0, cd1, cg0, cg1)


def kernel(x0, ln0_scale, ln0_bias, Wr, br, W1, b1, ln1_scale, ln1_bias, W2, b2):
    B, S, D = x0.shape
    E = Wr.shape[-1]
    H = W1.shape[-1]
    N = B * S
    top_k = 2
    cap = max(1, int(math.ceil(1.0 * N * top_k / E)))
    ec = E * cap

    xf = x0.reshape(N, D)
    wr_p = jnp.zeros((D, _EPAD), jnp.float32).at[:, :E].set(Wr)
    br_p = jnp.zeros((1, _EPAD), jnp.float32).at[0, :E].set(br)

    x12, sd0, sd1, cd0, cd1, cg0, cg1 = _route(
        xf, ln0_scale.reshape(1, D), ln0_bias.reshape(1, D), wr_p, br_p,
        blk=256, n_exp=E, cap=cap, ec=ec)

    ei = _dispatch(x12.reshape(N, D), sd0.reshape(N), sd1.reshape(N),
                   ec=ec, n=N, d=D)

    fold = D // 128
    y24 = _mlp(ei.reshape((ec + _TRASH) * fold, 128), W1, b1.reshape(E, 1, H),
               ln1_scale.reshape(E, 1, H), ln1_bias.reshape(E, 1, H),
               W2, b2.reshape(E, 1, D), n_exp=E, cap=cap, d=D, hdim=H)

    out = _combine(y24.reshape(ec, D), xf, cd0.reshape(N), cd1.reshape(N),
                   cg0, cg1, n=N, d=D)

    return out.reshape(B, S, D)


# speedup vs baseline: 1.2519x; 1.0471x over previous
import functools
import math

import jax
import jax.numpy as jnp
from jax import lax
from jax.experimental import pallas as pl
from jax.experimental.pallas import tpu as pltpu
from jax.experimental.pallas import tpu_sc as plsc

_NC = 2
_NS = 16
_LANES = 16
_NW = _NC * _NS

_EPAD = 128
_NEG = -1e30


def _route_body(x0_ref, s_ref, b_ref, wr_ref, br_ref,
                x_ref, sd0_ref, sd1_ref, cd0_ref, cd1_ref, cg0_ref, cg1_ref,
                carry_ref, *, blk, n_exp, cap, ec):
    b = pl.program_id(0)

    @pl.when(b == 0)
    def _():
        carry_ref[...] = jnp.zeros_like(carry_ref)

    x0 = x0_ref[...]
    mean = jnp.mean(x0, axis=1, keepdims=True)
    var = jnp.mean((x0 - mean) ** 2, axis=1, keepdims=True)
    x = (x0 - mean) * lax.rsqrt(var + 1e-6) * s_ref[...] + b_ref[...]
    x = jnp.maximum(x, 0.0)
    x_ref[...] = jnp.reshape(x, x_ref.shape)

    logits = jnp.dot(x, wr_ref[...], preferred_element_type=jnp.float32)
    logits = logits + br_ref[...]
    lane = lax.broadcasted_iota(jnp.int32, logits.shape, 1)
    logits = jnp.where(lane < n_exp, logits, _NEG)

    m1 = jnp.max(logits, axis=1, keepdims=True)
    e0 = jnp.min(jnp.where(logits == m1, lane, _EPAD), axis=1, keepdims=True)
    l2 = jnp.where(lane == e0, _NEG, logits)
    m2 = jnp.max(l2, axis=1, keepdims=True)
    e1 = jnp.min(jnp.where(l2 == m2, lane, _EPAD), axis=1, keepdims=True)
    g0 = 1.0 / (1.0 + jnp.exp(m2 - m1))
    g1 = 1.0 - g0

    oh0 = (lane == e0).astype(jnp.float32)
    oh1 = (lane == e1).astype(jnp.float32)
    oh = oh0 + oh1
    ri = lax.broadcasted_iota(jnp.int32, (blk, blk), 0)
    ci = lax.broadcasted_iota(jnp.int32, (blk, blk), 1)
    ltri = (ci < ri).astype(jnp.float32)
    before = jnp.dot(ltri, oh, preferred_element_type=jnp.float32)
    before = before + carry_ref[...]
    carry_ref[...] = carry_ref[...] + jnp.sum(oh, axis=0, keepdims=True)

    pos0 = jnp.sum(before * oh0, axis=1, keepdims=True).astype(jnp.int32)
    pos1 = jnp.sum((before + oh0) * oh1, axis=1, keepdims=True).astype(jnp.int32)
    tok = b * blk + lax.broadcasted_iota(jnp.int32, (blk, 1), 0)
    v0 = pos0 < cap
    v1 = pos1 < cap
    trash = ec + jnp.bitwise_and(tok, _TRASH - 1)
    sd0_ref[...] = jnp.where(v0, e0 * cap + pos0, trash)
    sd1_ref[...] = jnp.where(v1, e1 * cap + pos1, trash)
    cd0_ref[...] = e0 * cap + jnp.minimum(pos0, cap - 1)
    cd1_ref[...] = e1 * cap + jnp.minimum(pos1, cap - 1)
    cg0_ref[...] = jnp.broadcast_to(jnp.where(v0, g0, 0.0), (blk, _LANES))
    cg1_ref[...] = jnp.broadcast_to(jnp.where(v1, g1, 0.0), (blk, _LANES))


def _route(xf, ln0_scale, ln0_bias, wr_p, br_p, *, blk, n_exp, cap, ec):
    n, d = xf.shape
    grid = n // blk
    body = functools.partial(_route_body, blk=blk, n_exp=n_exp, cap=cap, ec=ec)
    col_i = jax.ShapeDtypeStruct((n, 1), jnp.int32)
    gate_f = jax.ShapeDtypeStruct((n, _LANES), jnp.float32)
    return pl.pallas_call(
        body,
        grid=(grid,),
        in_specs=[
            pl.BlockSpec((blk, d), lambda b: (b, 0)),
            pl.BlockSpec((1, d), lambda b: (0, 0)),
            pl.BlockSpec((1, d), lambda b: (0, 0)),
            pl.BlockSpec((d, _EPAD), lambda b: (0, 0)),
            pl.BlockSpec((1, _EPAD), lambda b: (0, 0)),
        ],
        out_specs=[
            pl.BlockSpec((blk * (d // 128), 128), lambda b: (b, 0)),
        ] + [pl.BlockSpec((blk, 1), lambda b: (b, 0))] * 4
          + [pl.BlockSpec((blk, _LANES), lambda b: (b, 0))] * 2,
        out_shape=[jax.ShapeDtypeStruct((n * (d // 128), 128), jnp.float32),
                   col_i, col_i, col_i, col_i, gate_f, gate_f],
        scratch_shapes=[pltpu.VMEM((1, _EPAD), jnp.float32)],
        compiler_params=pltpu.CompilerParams(
            dimension_semantics=("arbitrary",)),
    )(xf, ln0_scale, ln0_bias, wr_p, br_p)


_TRASH = 512


def _dispatch(xf, sd0, sd1, *, ec, n, d):
    tpw = n // _NW
    gch = 32
    nck = tpw // gch
    mesh = plsc.VectorSubcoreMesh(core_axis_name="c", subcore_axis_name="s")

    @functools.partial(
        pl.kernel,
        out_type=jax.ShapeDtypeStruct((ec + _TRASH, d), jnp.float32),
        mesh=mesh,
        scratch_types=[
            [pltpu.VMEM((gch,), jnp.int32)] * 2,
            [pltpu.VMEM((gch,), jnp.int32)] * 2,
            [pltpu.VMEM((gch, d), jnp.float32)] * 2,
            [pltpu.SemaphoreType.DMA] * 2,
            [pltpu.SemaphoreType.DMA] * 2,
            [pltpu.SemaphoreType.DMA] * 2,
        ],
        compiler_params=pltpu.CompilerParams(use_tc_tiling_on_sc=False),
    )
    def k(x_hbm, sd0_hbm, sd1_hbm, out_hbm, i0, i1, xbuf, xg, s0, s1):
        cid = lax.axis_index("c")
        sid = lax.axis_index("s")
        wid = sid * _NC + cid
        cps = [[None, None], [None, None], [None, None]]
        for c in range(nck):
            bb = c & 1
            if c >= 2:
                cps[1][bb].wait()
                cps[2][bb].wait()
            tokbase = wid * tpw + c * gch
            cps[0][bb] = pltpu.async_copy(
                x_hbm.at[pl.ds(tokbase, gch)], xbuf[bb], xg[bb])
            pltpu.sync_copy(sd0_hbm.at[pl.ds(tokbase, gch)], i0[bb])
            pltpu.sync_copy(sd1_hbm.at[pl.ds(tokbase, gch)], i1[bb])
            cps[0][bb].wait()
            cps[1][bb] = pltpu.async_copy(
                xbuf[bb], out_hbm.at[i0[bb]], s0[bb])
            cps[2][bb] = pltpu.async_copy(
                xbuf[bb], out_hbm.at[i1[bb]], s1[bb])
        for c in range(min(nck, 2)):
            cps[1][c].wait()
            cps[2][c].wait()

    return k(xf, sd0, sd1)


def _mlp_body(xe_ref, w1_ref, b1_ref, s1_ref, t1_ref, w2_ref, b2_ref, y_ref,
              *, cap, d):
    xe = jnp.reshape(xe_ref[...], (cap, d))
    h = jnp.dot(xe, w1_ref[0], preferred_element_type=jnp.float32) + b1_ref[0]
    mean = jnp.mean(h, axis=1, keepdims=True)
    var = jnp.mean((h - mean) ** 2, axis=1, keepdims=True)
    h = (h - mean) * lax.rsqrt(var + 1e-6) * s1_ref[0] + t1_ref[0]
    h = jnp.maximum(h, 0.0)
    y = jnp.dot(h, w2_ref[0], preferred_element_type=jnp.float32) + b2_ref[0]
    y_ref[...] = jnp.reshape(y, y_ref.shape)


def _mlp(ei28, w1, b1, s1, t1, w2, b2, *, n_exp, cap, d, hdim):
    fold = d // 128
    body = functools.partial(_mlp_body, cap=cap, d=d)
    return pl.pallas_call(
        body,
        grid=(n_exp,),
        in_specs=[
            pl.BlockSpec((cap * fold, 128), lambda e: (e, 0)),
            pl.BlockSpec((1, d, hdim), lambda e: (e, 0, 0)),
            pl.BlockSpec((1, 1, hdim), lambda e: (e, 0, 0)),
            pl.BlockSpec((1, 1, hdim), lambda e: (e, 0, 0)),
            pl.BlockSpec((1, 1, hdim), lambda e: (e, 0, 0)),
            pl.BlockSpec((1, hdim, d), lambda e: (e, 0, 0)),
            pl.BlockSpec((1, 1, d), lambda e: (e, 0, 0)),
        ],
        out_specs=pl.BlockSpec((cap * fold, 128), lambda e: (e, 0)),
        out_shape=jax.ShapeDtypeStruct((n_exp * cap * fold, 128), jnp.float32),
        compiler_params=pltpu.CompilerParams(
            dimension_semantics=("arbitrary",)),
    )(ei28, w1, b1, s1, t1, w2, b2)


def _combine(y, cd0, cd1, cg0, cg1, *, n, d):
    tpw = n // _NW
    ch = 16
    nch = tpw // ch
    nseg = d // _LANES
    mesh = plsc.VectorSubcoreMesh(core_axis_name="c", subcore_axis_name="s")

    @functools.partial(
        pl.kernel,
        out_type=jax.ShapeDtypeStruct((n, d), jnp.float32),
        mesh=mesh,
        scratch_types=[
            pltpu.VMEM((tpw,), jnp.int32),
            pltpu.VMEM((tpw,), jnp.int32),
            pltpu.VMEM((tpw, _LANES), jnp.float32),
            pltpu.VMEM((tpw, _LANES), jnp.float32),
            [pltpu.VMEM((ch, d), jnp.float32)] * 3,
            [pltpu.VMEM((ch, d), jnp.float32)] * 3,
            [pltpu.SemaphoreType.DMA] * 3,
            [pltpu.SemaphoreType.DMA] * 3,
            [pltpu.SemaphoreType.DMA] * 3,
        ],
        compiler_params=pltpu.CompilerParams(use_tc_tiling_on_sc=False),
    )
    def k(y_hbm, cd0_hbm, cd1_hbm, cg0_hbm, cg1_hbm, out_hbm,
          idx0, idx1, g0v, g1v, r0, r1, s0, s1, sw):
        cid = lax.axis_index("c")
        sid = lax.axis_index("s")
        wid = sid * _NC + cid
        base = wid * tpw
        pltpu.sync_copy(cd0_hbm.at[pl.ds(base, tpw)], idx0)
        pltpu.sync_copy(cd1_hbm.at[pl.ds(base, tpw)], idx1)
        pltpu.sync_copy(cg0_hbm.at[pl.ds(base, tpw)], g0v)
        pltpu.sync_copy(cg1_hbm.at[pl.ds(base, tpw)], g1v)

        nbuf = 3
        cp0 = [None] * nbuf
        cp1 = [None] * nbuf
        cpw = [None] * nbuf

        def stage(c):
            bb = c % nbuf
            cp0[bb] = pltpu.async_copy(
                y_hbm.at[idx0.at[pl.ds(c * ch, ch)]], r0[bb], s0[bb])
            cp1[bb] = pltpu.async_copy(
                y_hbm.at[idx1.at[pl.ds(c * ch, ch)]], r1[bb], s1[bb])

        def compute(c):
            bb = c % nbuf
            off = c * ch

            def row(j, _):
                gb0 = g0v[off + j, :]
                gb1 = g1v[off + j, :]
                for q in range(nseg):
                    seg = pl.ds(q * _LANES, _LANES)
                    r0[bb][j, seg] = (gb0 * r0[bb][j, seg]
                                      + gb1 * r1[bb][j, seg])
                return 0

            lax.fori_loop(0, ch, row, 0)

        stage(0)
        stage(1)
        for c in range(nch):
            bb = c % nbuf
            if c + 2 < nch:
                nb = (c + 2) % nbuf
                if c >= 1:
                    cpw[nb].wait()
                stage(c + 2)
            cp0[bb].wait()
            cp1[bb].wait()
            compute(c)
            cpw[bb] = pltpu.async_copy(
                r0[bb], out_hbm.at[pl.ds(base + c * ch, ch)], sw[bb])
        for c in range(max(0, nch - nbuf), nch):
            cpw[c % nbuf].wait()

    return k(y, cd0, cd1, cg0, cg1)


def _residual_body(x0_ref, mix_ref, out_ref, *, blk, d):
    out_ref[...] = x0_ref[...] + jnp.reshape(mix_ref[...], (blk, d))


def _residual(xf, mix12, *, n, d, blk):
    fold = d // 128
    body = functools.partial(_residual_body, blk=blk, d=d)
    return pl.pallas_call(
        body,
        grid=(n // blk,),
        in_specs=[
            pl.BlockSpec((blk, d), lambda b: (b, 0)),
            pl.BlockSpec((blk * fold, 128), lambda b: (b, 0)),
        ],
        out_specs=pl.BlockSpec((blk, d), lambda b: (b, 0)),
        out_shape=jax.ShapeDtypeStruct((n, d), jnp.float32),
    )(xf, mix12)


def kernel(x0, ln0_scale, ln0_bias, Wr, br, W1, b1, ln1_scale, ln1_bias, W2, b2):
    B, S, D = x0.shape
    E = Wr.shape[-1]
    H = W1.shape[-1]
    N = B * S
    top_k = 2
    cap = max(1, int(math.ceil(1.0 * N * top_k / E)))
    ec = E * cap

    xf = x0.reshape(N, D)
    wr_p = jnp.zeros((D, _EPAD), jnp.float32).at[:, :E].set(Wr)
    br_p = jnp.zeros((1, _EPAD), jnp.float32).at[0, :E].set(br)

    x12, sd0, sd1, cd0, cd1, cg0, cg1 = _route(
        xf, ln0_scale.reshape(1, D), ln0_bias.reshape(1, D), wr_p, br_p,
        blk=256, n_exp=E, cap=cap, ec=ec)

    ei = _dispatch(x12.reshape(N, D), sd0.reshape(N), sd1.reshape(N),
                   ec=ec, n=N, d=D)

    fold = D // 128
    y24 = _mlp(ei.reshape((ec + _TRASH) * fold, 128), W1, b1.reshape(E, 1, H),
               ln1_scale.reshape(E, 1, H), ln1_bias.reshape(E, 1, H),
               W2, b2.reshape(E, 1, D), n_exp=E, cap=cap, d=D, hdim=H)

    mix = _combine(y24.reshape(ec, D), cd0.reshape(N), cd1.reshape(N),
                   cg0, cg1, n=N, d=D)
    out = _residual(xf, mix.reshape(N * fold, 128), n=N, d=D, blk=256)

    return out.reshape(B, S, D)


# speedup vs baseline: 1.2751x; 1.0185x over previous
import functools
import math

import jax
import jax.numpy as jnp
from jax import lax
from jax.experimental import pallas as pl
from jax.experimental.pallas import tpu as pltpu
from jax.experimental.pallas import tpu_sc as plsc

_NC = 2
_NS = 16
_LANES = 16
_NW = _NC * _NS

_EPAD = 128
_NEG = -1e30


def _route_body(x0_ref, s_ref, b_ref, wr_ref, br_ref,
                x_ref, sd0_ref, sd1_ref, cd0_ref, cd1_ref, cg0_ref, cg1_ref,
                carry_ref, *, blk, n_exp, cap, ec):
    b = pl.program_id(0)

    @pl.when(b == 0)
    def _():
        carry_ref[...] = jnp.zeros_like(carry_ref)

    x0 = x0_ref[...]
    mean = jnp.mean(x0, axis=1, keepdims=True)
    var = jnp.mean((x0 - mean) ** 2, axis=1, keepdims=True)
    x = (x0 - mean) * lax.rsqrt(var + 1e-6) * s_ref[...] + b_ref[...]
    x = jnp.maximum(x, 0.0)
    x_ref[...] = jnp.reshape(x, x_ref.shape)

    logits = jnp.dot(x, wr_ref[...], preferred_element_type=jnp.float32)
    logits = logits + br_ref[...]
    lane = lax.broadcasted_iota(jnp.int32, logits.shape, 1)
    logits = jnp.where(lane < n_exp, logits, _NEG)

    m1 = jnp.max(logits, axis=1, keepdims=True)
    e0 = jnp.min(jnp.where(logits == m1, lane, _EPAD), axis=1, keepdims=True)
    l2 = jnp.where(lane == e0, _NEG, logits)
    m2 = jnp.max(l2, axis=1, keepdims=True)
    e1 = jnp.min(jnp.where(l2 == m2, lane, _EPAD), axis=1, keepdims=True)
    g0 = 1.0 / (1.0 + jnp.exp(m2 - m1))
    g1 = 1.0 - g0

    oh0 = (lane == e0).astype(jnp.float32)
    oh1 = (lane == e1).astype(jnp.float32)
    oh = oh0 + oh1
    ri = lax.broadcasted_iota(jnp.int32, (blk, blk), 0)
    ci = lax.broadcasted_iota(jnp.int32, (blk, blk), 1)
    ltri = (ci < ri).astype(jnp.float32)
    before = jnp.dot(ltri, oh, preferred_element_type=jnp.float32)
    before = before + carry_ref[...]
    carry_ref[...] = carry_ref[...] + jnp.sum(oh, axis=0, keepdims=True)

    pos0 = jnp.sum(before * oh0, axis=1, keepdims=True).astype(jnp.int32)
    pos1 = jnp.sum((before + oh0) * oh1, axis=1, keepdims=True).astype(jnp.int32)
    tok = b * blk + lax.broadcasted_iota(jnp.int32, (blk, 1), 0)
    v0 = pos0 < cap
    v1 = pos1 < cap
    trash = ec + jnp.bitwise_and(tok, _TRASH - 1)
    sd0_ref[...] = jnp.where(v0, e0 * cap + pos0, trash)
    sd1_ref[...] = jnp.where(v1, e1 * cap + pos1, trash)
    cd0_ref[...] = e0 * cap + jnp.minimum(pos0, cap - 1)
    cd1_ref[...] = e1 * cap + jnp.minimum(pos1, cap - 1)
    cg0_ref[...] = jnp.broadcast_to(jnp.where(v0, g0, 0.0), (blk, _LANES))
    cg1_ref[...] = jnp.broadcast_to(jnp.where(v1, g1, 0.0), (blk, _LANES))


def _route(xf, ln0_scale, ln0_bias, wr_p, br_p, *, blk, n_exp, cap, ec):
    n, d = xf.shape
    grid = n // blk
    body = functools.partial(_route_body, blk=blk, n_exp=n_exp, cap=cap, ec=ec)
    col_i = jax.ShapeDtypeStruct((n, 1), jnp.int32)
    gate_f = jax.ShapeDtypeStruct((n, _LANES), jnp.float32)
    return pl.pallas_call(
        body,
        grid=(grid,),
        in_specs=[
            pl.BlockSpec((blk, d), lambda b: (b, 0)),
            pl.BlockSpec((1, d), lambda b: (0, 0)),
            pl.BlockSpec((1, d), lambda b: (0, 0)),
            pl.BlockSpec((d, _EPAD), lambda b: (0, 0)),
            pl.BlockSpec((1, _EPAD), lambda b: (0, 0)),
        ],
        out_specs=[
            pl.BlockSpec((blk * (d // 128), 128), lambda b: (b, 0)),
        ] + [pl.BlockSpec((blk, 1), lambda b: (b, 0))] * 4
          + [pl.BlockSpec((blk, _LANES), lambda b: (b, 0))] * 2,
        out_shape=[jax.ShapeDtypeStruct((n * (d // 128), 128), jnp.float32),
                   col_i, col_i, col_i, col_i, gate_f, gate_f],
        scratch_shapes=[pltpu.VMEM((1, _EPAD), jnp.float32)],
        compiler_params=pltpu.CompilerParams(
            dimension_semantics=("arbitrary",)),
    )(xf, ln0_scale, ln0_bias, wr_p, br_p)


_TRASH = 512


def _dispatch(xf, sd0, sd1, *, ec, n, d):
    tpw = n // _NW
    gch = 32
    nck = tpw // gch
    mesh = plsc.VectorSubcoreMesh(core_axis_name="c", subcore_axis_name="s")

    @functools.partial(
        pl.kernel,
        out_type=jax.ShapeDtypeStruct((ec + _TRASH, d), jnp.float32),
        mesh=mesh,
        scratch_types=[
            [pltpu.VMEM((gch,), jnp.int32)] * 2,
            [pltpu.VMEM((gch,), jnp.int32)] * 2,
            [pltpu.VMEM((gch, d), jnp.float32)] * 2,
            [pltpu.SemaphoreType.DMA] * 2,
            [pltpu.SemaphoreType.DMA] * 2,
            [pltpu.SemaphoreType.DMA] * 2,
        ],
        compiler_params=pltpu.CompilerParams(use_tc_tiling_on_sc=False),
    )
    def k(x_hbm, sd0_hbm, sd1_hbm, out_hbm, i0, i1, xbuf, xg, s0, s1):
        cid = lax.axis_index("c")
        sid = lax.axis_index("s")
        wid = sid * _NC + cid
        cps = [[None, None], [None, None], [None, None]]
        for c in range(nck):
            bb = c & 1
            if c >= 2:
                cps[1][bb].wait()
                cps[2][bb].wait()
            tokbase = wid * tpw + c * gch
            cps[0][bb] = pltpu.async_copy(
                x_hbm.at[pl.ds(tokbase, gch)], xbuf[bb], xg[bb])
            pltpu.sync_copy(sd0_hbm.at[pl.ds(tokbase, gch)], i0[bb])
            pltpu.sync_copy(sd1_hbm.at[pl.ds(tokbase, gch)], i1[bb])
            cps[0][bb].wait()
            cps[1][bb] = pltpu.async_copy(
                xbuf[bb], out_hbm.at[i0[bb]], s0[bb])
            cps[2][bb] = pltpu.async_copy(
                xbuf[bb], out_hbm.at[i1[bb]], s1[bb])
        for c in range(min(nck, 2)):
            cps[1][c].wait()
            cps[2][c].wait()

    return k(xf, sd0, sd1)


def _mlp_body(xe_ref, w1_ref, b1_ref, s1_ref, t1_ref, w2_ref, b2_ref, y_ref,
              *, cap, d):
    xe = jnp.reshape(xe_ref[...], (cap, d))
    h = jnp.dot(xe, w1_ref[0], preferred_element_type=jnp.float32) + b1_ref[0]
    mean = jnp.mean(h, axis=1, keepdims=True)
    var = jnp.mean((h - mean) ** 2, axis=1, keepdims=True)
    h = (h - mean) * lax.rsqrt(var + 1e-6) * s1_ref[0] + t1_ref[0]
    h = jnp.maximum(h, 0.0)
    y = jnp.dot(h, w2_ref[0], preferred_element_type=jnp.float32) + b2_ref[0]
    y_ref[...] = jnp.reshape(y, y_ref.shape)


def _mlp(ei28, w1, b1, s1, t1, w2, b2, *, n_exp, cap, d, hdim):
    fold = d // 128
    body = functools.partial(_mlp_body, cap=cap, d=d)
    return pl.pallas_call(
        body,
        grid=(n_exp,),
        in_specs=[
            pl.BlockSpec((cap * fold, 128), lambda e: (e, 0)),
            pl.BlockSpec((1, d, hdim), lambda e: (e, 0, 0)),
            pl.BlockSpec((1, 1, hdim), lambda e: (e, 0, 0)),
            pl.BlockSpec((1, 1, hdim), lambda e: (e, 0, 0)),
            pl.BlockSpec((1, 1, hdim), lambda e: (e, 0, 0)),
            pl.BlockSpec((1, hdim, d), lambda e: (e, 0, 0)),
            pl.BlockSpec((1, 1, d), lambda e: (e, 0, 0)),
        ],
        out_specs=pl.BlockSpec((cap * fold, 128), lambda e: (e, 0)),
        out_shape=jax.ShapeDtypeStruct((n_exp * cap * fold, 128), jnp.float32),
        compiler_params=pltpu.CompilerParams(
            dimension_semantics=("arbitrary",)),
    )(ei28, w1, b1, s1, t1, w2, b2)


def _combine(y, cd0, cd1, cg0, cg1, *, n, d):
    tpw = n // _NW
    ch = 16
    nch = tpw // ch
    nseg = d // _LANES
    mesh = plsc.VectorSubcoreMesh(core_axis_name="c", subcore_axis_name="s")

    @functools.partial(
        pl.kernel,
        out_type=jax.ShapeDtypeStruct((n, d), jnp.float32),
        mesh=mesh,
        scratch_types=[
            pltpu.VMEM((tpw,), jnp.int32),
            pltpu.VMEM((tpw,), jnp.int32),
            pltpu.VMEM((tpw, _LANES), jnp.float32),
            pltpu.VMEM((tpw, _LANES), jnp.float32),
            [pltpu.VMEM((ch, d), jnp.float32)] * 3,
            [pltpu.VMEM((ch, d), jnp.float32)] * 3,
            [pltpu.SemaphoreType.DMA] * 3,
            [pltpu.SemaphoreType.DMA] * 3,
            [pltpu.SemaphoreType.DMA] * 3,
        ],
        compiler_params=pltpu.CompilerParams(use_tc_tiling_on_sc=False),
    )
    def k(y_hbm, cd0_hbm, cd1_hbm, cg0_hbm, cg1_hbm, out_hbm,
          idx0, idx1, g0v, g1v, r0, r1, s0, s1, sw):
        cid = lax.axis_index("c")
        sid = lax.axis_index("s")
        wid = sid * _NC + cid
        base = wid * tpw
        pltpu.sync_copy(cd0_hbm.at[pl.ds(base, tpw)], idx0)
        pltpu.sync_copy(cd1_hbm.at[pl.ds(base, tpw)], idx1)
        pltpu.sync_copy(cg0_hbm.at[pl.ds(base, tpw)], g0v)
        pltpu.sync_copy(cg1_hbm.at[pl.ds(base, tpw)], g1v)

        nbuf = 3
        cp0 = [None] * nbuf
        cp1 = [None] * nbuf
        cpw = [None] * nbuf

        def stage(c):
            bb = c % nbuf
            cp0[bb] = pltpu.async_copy(
                y_hbm.at[idx0.at[pl.ds(c * ch, ch)]], r0[bb], s0[bb])
            cp1[bb] = pltpu.async_copy(
                y_hbm.at[idx1.at[pl.ds(c * ch, ch)]], r1[bb], s1[bb])

        def compute(c):
            bb = c % nbuf
            off = c * ch

            def row(j, _):
                gb0 = g0v[off + j, :]
                gb1 = g1v[off + j, :]
                for q in range(nseg):
                    seg = pl.ds(q * _LANES, _LANES)
                    r0[bb][j, seg] = (gb0 * r0[bb][j, seg]
                                      + gb1 * r1[bb][j, seg])
                return 0

            lax.fori_loop(0, ch, row, 0)

        stage(0)
        stage(1)
        for c in range(nch):
            bb = c % nbuf
            if c + 2 < nch:
                nb = (c + 2) % nbuf
                if c >= 1:
                    cpw[nb].wait()
                stage(c + 2)
            cp0[bb].wait()
            cp1[bb].wait()
            compute(c)
            cpw[bb] = pltpu.async_copy(
                r0[bb], out_hbm.at[pl.ds(base + c * ch, ch)], sw[bb])
        for c in range(max(0, nch - nbuf), nch):
            cpw[c % nbuf].wait()

    return k(y, cd0, cd1, cg0, cg1)


def _residual_body(x0_ref, mix_ref, out_ref, *, blk, d):
    out_ref[...] = x0_ref[...] + jnp.reshape(mix_ref[...], (blk, d))


def _residual(xf, mix12, *, n, d, blk):
    fold = d // 128
    body = functools.partial(_residual_body, blk=blk, d=d)
    return pl.pallas_call(
        body,
        grid=(n // blk,),
        in_specs=[
            pl.BlockSpec((blk, d), lambda b: (b, 0)),
            pl.BlockSpec((blk * fold, 128), lambda b: (b, 0)),
        ],
        out_specs=pl.BlockSpec((blk, d), lambda b: (b, 0)),
        out_shape=jax.ShapeDtypeStruct((n, d), jnp.float32),
    )(xf, mix12)


def kernel(x0, ln0_scale, ln0_bias, Wr, br, W1, b1, ln1_scale, ln1_bias, W2, b2):
    B, S, D = x0.shape
    E = Wr.shape[-1]
    H = W1.shape[-1]
    N = B * S
    top_k = 2
    cap = max(1, int(math.ceil(1.0 * N * top_k / E)))
    ec = E * cap

    xf = x0.reshape(N, D)
    wr_p = jnp.zeros((D, _EPAD), jnp.float32).at[:, :E].set(Wr)
    br_p = jnp.zeros((1, _EPAD), jnp.float32).at[0, :E].set(br)

    x12, sd0, sd1, cd0, cd1, cg0, cg1 = _route(
        xf, ln0_scale.reshape(1, D), ln0_bias.reshape(1, D), wr_p, br_p,
        blk=512, n_exp=E, cap=cap, ec=ec)

    ei = _dispatch(x12.reshape(N, D), sd0.reshape(N), sd1.reshape(N),
                   ec=ec, n=N, d=D)

    fold = D // 128
    y24 = _mlp(ei.reshape((ec + _TRASH) * fold, 128), W1, b1.reshape(E, 1, H),
               ln1_scale.reshape(E, 1, H), ln1_bias.reshape(E, 1, H),
               W2, b2.reshape(E, 1, D), n_exp=E, cap=cap, d=D, hdim=H)

    mix = _combine(y24.reshape(ec, D), cd0.reshape(N), cd1.reshape(N),
                   cg0, cg1, n=N, d=D)
    out = _residual(xf, mix.reshape(N * fold, 128), n=N, d=D, blk=256)

    return out.reshape(B, S, D)
